# Initial kernel scaffold; baseline (speedup 1.0000x reference)
import jax
import jax.numpy as jnp
from jax.experimental import pallas as pl


def kernel(h, r_flat, t_flat, segment_ids, H_table, R_table, T_table):
    raise NotImplementedError("write your pallas kernel here")


# speedup vs baseline: 33.1386x; 33.1386x over previous
import functools

import jax
import jax.numpy as jnp
from jax import lax
from jax.experimental import pallas as pl
from jax.experimental.pallas import tpu as pltpu
from jax.experimental.pallas import tpu_sc as plsc

L = 16
NS = 16
ROW = 128


def _precompute_body(h_ref, r_ref, t_ref, m_ref, ts_ref, rs_ref):
    hmat = h_ref[...]
    rmat = r_ref[...]
    tmat = t_ref[...]
    m_ref[...] = lax.dot_general(
        hmat, rmat, (((1,), (1,)), ((), ())),
        preferred_element_type=jnp.float32)
    ts_ref[...] = jnp.sum(tmat, axis=1, keepdims=True)
    rs_ref[...] = jnp.sum(rmat, axis=1, keepdims=True)


def _precompute(h_table, r_pad, t_pad):
    nh = h_table.shape[0]
    nrp = r_pad.shape[0]
    ntp = t_pad.shape[0]
    return pl.pallas_call(
        _precompute_body,
        out_shape=(
            jax.ShapeDtypeStruct((nh, nrp), jnp.float32),
            jax.ShapeDtypeStruct((ntp, 1), jnp.float32),
            jax.ShapeDtypeStruct((nrp, 1), jnp.float32),
        ),
    )(h_table, r_pad, t_pad)


def _sc_kernel(n_tok, b, ntp, nrp, nflat,
               seg_hbm, r_hbm, t_hbm, h_hbm, m_hbm, tsum_hbm, rsum_hbm,
               v_hbm,
               h_v, tsum_v, rsum_v, seg_v, r_v, t_v,
               idx_row, s_row, ex_row, exd_row,
               den_v, num_v, v_v,
               den_sp, num_sp):
    wid = lax.axis_index("s")
    rows_per_w = n_tok // (NS * ROW)
    bpw = b // NS
    row0 = wid * rows_per_w

    pltpu.sync_copy(seg_hbm.at[pl.ds(row0, rows_per_w)], seg_v)
    pltpu.sync_copy(r_hbm.at[pl.ds(row0, rows_per_w)], r_v)
    pltpu.sync_copy(t_hbm.at[pl.ds(row0, rows_per_w)], t_v)
    pltpu.sync_copy(h_hbm, h_v)
    pltpu.sync_copy(tsum_hbm, tsum_v)
    pltpu.sync_copy(rsum_hbm, rsum_v)

    @pl.loop(0, bpw // L)
    def _zero(k):
        den_v[pl.ds(k * L, L)] = jnp.zeros((L,), jnp.float32)

    pltpu.sync_copy(den_v, den_sp.at[pl.ds(wid * bpw, bpw)])
    pltpu.sync_copy(den_v, num_sp.at[pl.ds(wid * bpw, bpw)])
    plsc.subcore_barrier()

    @pl.loop(0, rows_per_w)
    def _row(j):
        for u in range(ROW // L):
            sl = pl.ds(u * L, L)
            seg16 = seg_v[j, sl]
            r16 = r_v[j, sl]
            h16 = plsc.load_gather(h_v, [seg16])
            idx_row[sl] = h16 * nrp + r16
        pltpu.sync_copy(m_hbm.at[idx_row], s_row)
        for u in range(ROW // L):
            sl = pl.ds(u * L, L)
            s16 = s_row[sl]
            t16 = t_v[j, sl]
            r16 = r_v[j, sl]
            d16 = plsc.load_gather(tsum_v, [t16]) - plsc.load_gather(rsum_v, [r16])
            ex16 = jnp.exp(s16)
            ex_row[sl] = ex16
            exd_row[sl] = ex16 * d16
        pltpu.sync_copy(ex_row, den_sp.at[seg_v.at[j]], add=True)
        pltpu.sync_copy(exd_row, num_sp.at[seg_v.at[j]], add=True)

    plsc.subcore_barrier()

    pltpu.sync_copy(den_sp.at[pl.ds(wid * bpw, bpw)], den_v)
    pltpu.sync_copy(num_sp.at[pl.ds(wid * bpw, bpw)], num_v)

    @pl.loop(0, bpw // L)
    def _div(k):
        sl = pl.ds(k * L, L)
        d16 = den_v[sl]
        n16 = num_v[sl]
        v_v[sl] = jnp.where(d16 > 0.0, n16 / d16, 0.0)

    pltpu.sync_copy(v_v, v_hbm.at[pl.ds(wid * bpw, bpw)])


def _sc_run(seg2, r2, t2, h, m_flat, tsum, rsum):
    n_rows = seg2.shape[0]
    n_tok = n_rows * ROW
    b = h.shape[0]
    ntp = tsum.shape[0]
    nrp = rsum.shape[0]
    nflat = m_flat.shape[0]
    bpw = b // NS
    mesh = plsc.VectorSubcoreMesh(
        core_axis_name="c", subcore_axis_name="s", num_cores=1)
    rows_per_w = n_rows // NS
    grid_kernel = pl.kernel(
        functools.partial(_sc_kernel, n_tok, b, ntp, nrp, nflat),
        out_type=jax.ShapeDtypeStruct((b,), jnp.float32),
        mesh=mesh,
        compiler_params=pltpu.CompilerParams(needs_layout_passes=False),
        scratch_types=[
            pltpu.VMEM((b,), jnp.int32),
            pltpu.VMEM((ntp,), jnp.float32),
            pltpu.VMEM((nrp,), jnp.float32),
            pltpu.VMEM((rows_per_w, ROW), jnp.int32),
            pltpu.VMEM((rows_per_w, ROW), jnp.int32),
            pltpu.VMEM((rows_per_w, ROW), jnp.int32),
            pltpu.VMEM((ROW,), jnp.int32),
            pltpu.VMEM((ROW,), jnp.float32),
            pltpu.VMEM((ROW,), jnp.float32),
            pltpu.VMEM((ROW,), jnp.float32),
            pltpu.VMEM((bpw,), jnp.float32),
            pltpu.VMEM((bpw,), jnp.float32),
            pltpu.VMEM((bpw,), jnp.float32),
            pltpu.VMEM_SHARED((b,), jnp.float32),
            pltpu.VMEM_SHARED((b,), jnp.float32),
        ],
    )
    return grid_kernel(seg2, r2, t2, h, m_flat, tsum, rsum)


def kernel(h, r_flat, t_flat, segment_ids, H_table, R_table, T_table):
    b = h.shape[0]
    n = segment_ids.shape[0]
    dim = H_table.shape[1]
    nr = R_table.shape[0]
    nt = T_table.shape[0]
    nrp = 64
    ntp = (nt + 7) // 8 * 8

    r_pad = jnp.pad(R_table, ((0, nrp - nr), (0, 0)))
    t_pad = jnp.pad(T_table, ((0, ntp - nt), (0, 0)))
    m, ts, rs = _precompute(H_table, r_pad, t_pad)

    seg2 = segment_ids.reshape(n // ROW, ROW)
    r2 = r_flat.reshape(n // ROW, ROW)
    t2 = t_flat.reshape(n // ROW, ROW)
    v = _sc_run(seg2, r2, t2, h, m.reshape(-1), ts.reshape(-1), rs.reshape(-1))
    return jnp.broadcast_to(v[:, None], (b, dim))


# speedup vs baseline: 60.2162x; 1.8171x over previous
import functools

import jax
import jax.numpy as jnp
from jax import lax
from jax.experimental import pallas as pl
from jax.experimental.pallas import tpu as pltpu
from jax.experimental.pallas import tpu_sc as plsc

L = 16
NS = 16
ROW = 128


def _precompute_body(h_ref, r_ref, t_ref, m_ref, ts_ref, rs_ref):
    hmat = h_ref[...]
    rmat = r_ref[...]
    tmat = t_ref[...]
    m_ref[...] = lax.dot_general(
        hmat, rmat, (((1,), (1,)), ((), ())),
        preferred_element_type=jnp.float32)
    ts_ref[...] = jnp.sum(tmat, axis=1, keepdims=True)
    rs_ref[...] = jnp.sum(rmat, axis=1, keepdims=True)


def _precompute(h_table, r_pad, t_pad):
    nh = h_table.shape[0]
    nrp = r_pad.shape[0]
    ntp = t_pad.shape[0]
    return pl.pallas_call(
        _precompute_body,
        out_shape=(
            jax.ShapeDtypeStruct((nh, nrp), jnp.float32),
            jax.ShapeDtypeStruct((ntp, 1), jnp.float32),
            jax.ShapeDtypeStruct((nrp, 1), jnp.float32),
        ),
    )(h_table, r_pad, t_pad)


def _sc_kernel(n_tok, b, ntp, nrp, nflat,
               seg_hbm, r_hbm, t_hbm, h_hbm, m_hbm, tsum_hbm, rsum_hbm,
               v_hbm,
               h_v, tsum_v, rsum_v, seg_v, r_v, t_v,
               idx_v, s_v, ex_v, exd_v,
               den_v, num_v, v_v,
               den_sp, num_sp,
               sem_in, sem_ga, sem_gb, sem_s):
    wid = lax.axis_index("s")
    rows_per_w = n_tok // (NS * ROW)
    bpw = b // NS
    row0 = wid * rows_per_w
    grp = 8
    ngrp = rows_per_w // grp

    in_copies = [
        (seg_hbm.at[pl.ds(row0, rows_per_w)], seg_v),
        (r_hbm.at[pl.ds(row0, rows_per_w)], r_v),
        (t_hbm.at[pl.ds(row0, rows_per_w)], t_v),
        (h_hbm, h_v),
        (tsum_hbm, tsum_v),
        (rsum_hbm, rsum_v),
    ]
    for src, dst in in_copies:
        pltpu.async_copy(src, dst, sem_in)

    @pl.loop(0, bpw // L)
    def _zero(k):
        den_v[pl.ds(k * L, L)] = jnp.zeros((L,), jnp.float32)

    for src, dst in in_copies:
        pltpu.make_async_copy(src, dst, sem_in).wait()

    pltpu.sync_copy(den_v, den_sp.at[pl.ds(wid * bpw, bpw)])
    pltpu.sync_copy(den_v, num_sp.at[pl.ds(wid * bpw, bpw)])
    plsc.subcore_barrier()

    @pl.loop(0, rows_per_w)
    def _idx(j):
        for u in range(ROW // L):
            sl = pl.ds(u * L, L)
            seg16 = seg_v[j, sl]
            r16 = r_v[j, sl]
            h16 = plsc.load_gather(h_v, [seg16])
            idx_v[j, sl] = h16 * nrp + r16

    gsems = (sem_ga, sem_gb)

    def _fire_gathers(g, sem):
        @pl.loop(g * grp, g * grp + grp)
        def _f(j):
            pltpu.async_copy(m_hbm.at[idx_v.at[j]], s_v.at[j], sem)

    def _drain_gathers(g, sem):
        @pl.loop(g * grp, g * grp + grp)
        def _d(j):
            pltpu.make_async_copy(m_hbm.at[idx_v.at[j]], s_v.at[j], sem).wait()

    _fire_gathers(0, gsems[0])
    for g in range(ngrp):
        if g + 1 < ngrp:
            _fire_gathers(g + 1, gsems[(g + 1) % 2])
        _drain_gathers(g, gsems[g % 2])

        @pl.loop(g * grp, g * grp + grp)
        def _compute(j):
            for u in range(ROW // L):
                sl = pl.ds(u * L, L)
                s16 = s_v[j, sl]
                t16 = t_v[j, sl]
                r16 = r_v[j, sl]
                d16 = (plsc.load_gather(tsum_v, [t16])
                       - plsc.load_gather(rsum_v, [r16]))
                ex16 = jnp.exp(s16)
                ex_v[j, sl] = ex16
                exd_v[j, sl] = ex16 * d16

        @pl.loop(g * grp, g * grp + grp)
        def _scatter(j):
            pltpu.async_copy(ex_v.at[j], den_sp.at[seg_v.at[j]], sem_s,
                             add=True)
            pltpu.async_copy(exd_v.at[j], num_sp.at[seg_v.at[j]], sem_s,
                             add=True)

    @pl.loop(0, rows_per_w)
    def _drain_s(j):
        pltpu.make_async_copy(ex_v.at[j], den_sp.at[seg_v.at[j]],
                              sem_s).wait()
        pltpu.make_async_copy(exd_v.at[j], num_sp.at[seg_v.at[j]],
                              sem_s).wait()

    plsc.subcore_barrier()

    pltpu.sync_copy(den_sp.at[pl.ds(wid * bpw, bpw)], den_v)
    pltpu.sync_copy(num_sp.at[pl.ds(wid * bpw, bpw)], num_v)

    @pl.loop(0, bpw // L)
    def _div(k):
        sl = pl.ds(k * L, L)
        d16 = den_v[sl]
        n16 = num_v[sl]
        v_v[sl] = jnp.where(d16 > 0.0, n16 / d16, 0.0)

    pltpu.sync_copy(v_v, v_hbm.at[pl.ds(wid * bpw, bpw)])


def _sc_run(seg2, r2, t2, h, m_flat, tsum, rsum):
    n_rows = seg2.shape[0]
    n_tok = n_rows * ROW
    b = h.shape[0]
    ntp = tsum.shape[0]
    nrp = rsum.shape[0]
    nflat = m_flat.shape[0]
    bpw = b // NS
    mesh = plsc.VectorSubcoreMesh(
        core_axis_name="c", subcore_axis_name="s", num_cores=1)
    rows_per_w = n_rows // NS
    grid_kernel = pl.kernel(
        functools.partial(_sc_kernel, n_tok, b, ntp, nrp, nflat),
        out_type=jax.ShapeDtypeStruct((b,), jnp.float32),
        mesh=mesh,
        compiler_params=pltpu.CompilerParams(needs_layout_passes=False),
        scratch_types=[
            pltpu.VMEM((b,), jnp.int32),
            pltpu.VMEM((ntp,), jnp.float32),
            pltpu.VMEM((nrp,), jnp.float32),
            pltpu.VMEM((rows_per_w, ROW), jnp.int32),
            pltpu.VMEM((rows_per_w, ROW), jnp.int32),
            pltpu.VMEM((rows_per_w, ROW), jnp.int32),
            pltpu.VMEM((rows_per_w, ROW), jnp.int32),
            pltpu.VMEM((rows_per_w, ROW), jnp.float32),
            pltpu.VMEM((rows_per_w, ROW), jnp.float32),
            pltpu.VMEM((rows_per_w, ROW), jnp.float32),
            pltpu.VMEM((bpw,), jnp.float32),
            pltpu.VMEM((bpw,), jnp.float32),
            pltpu.VMEM((bpw,), jnp.float32),
            pltpu.VMEM_SHARED((b,), jnp.float32),
            pltpu.VMEM_SHARED((b,), jnp.float32),
            pltpu.SemaphoreType.DMA,
            pltpu.SemaphoreType.DMA,
            pltpu.SemaphoreType.DMA,
            pltpu.SemaphoreType.DMA,
        ],
    )
    return grid_kernel(seg2, r2, t2, h, m_flat, tsum, rsum)


def kernel(h, r_flat, t_flat, segment_ids, H_table, R_table, T_table):
    b = h.shape[0]
    n = segment_ids.shape[0]
    dim = H_table.shape[1]
    nr = R_table.shape[0]
    nt = T_table.shape[0]
    nrp = 64
    ntp = (nt + 7) // 8 * 8

    r_pad = jnp.pad(R_table, ((0, nrp - nr), (0, 0)))
    t_pad = jnp.pad(T_table, ((0, ntp - nt), (0, 0)))
    m, ts, rs = _precompute(H_table, r_pad, t_pad)

    seg2 = segment_ids.reshape(n // ROW, ROW)
    r2 = r_flat.reshape(n // ROW, ROW)
    t2 = t_flat.reshape(n // ROW, ROW)
    v = _sc_run(seg2, r2, t2, h, m.reshape(-1), ts.reshape(-1), rs.reshape(-1))
    return jnp.broadcast_to(v[:, None], (b, dim))


# speedup vs baseline: 62.9651x; 1.0457x over previous
import functools

import jax
import jax.numpy as jnp
from jax import lax
from jax.experimental import pallas as pl
from jax.experimental.pallas import tpu as pltpu
from jax.experimental.pallas import tpu_sc as plsc

L = 16
NC = 2
NS = 16
ROW = 128


def _precompute_body(h_ref, r_ref, t_ref, m_ref, ts_ref, rs_ref):
    hmat = h_ref[...]
    rmat = r_ref[...]
    tmat = t_ref[...]
    m_ref[...] = lax.dot_general(
        hmat, rmat, (((1,), (1,)), ((), ())),
        preferred_element_type=jnp.float32)
    ts_ref[...] = jnp.sum(tmat, axis=1, keepdims=True)
    rs_ref[...] = jnp.sum(rmat, axis=1, keepdims=True)


def _precompute(h_table, r_pad, t_table):
    nh = h_table.shape[0]
    nrp = r_pad.shape[0]
    nt = t_table.shape[0]
    return pl.pallas_call(
        _precompute_body,
        out_shape=(
            jax.ShapeDtypeStruct((nh, nrp), jnp.float32),
            jax.ShapeDtypeStruct((nt, 1), jnp.float32),
            jax.ShapeDtypeStruct((nrp, 1), jnp.float32),
        ),
    )(h_table, r_pad, t_table)


def _combine_body(acc_ref, v_ref):
    a = acc_ref[...]
    den = a[0:1, :] + a[2:3, :]
    num = a[1:2, :] + a[3:4, :]
    v_ref[...] = jnp.where(den > 0.0, num / den, 0.0)


def _combine(acc):
    b = acc.shape[1]
    return pl.pallas_call(
        _combine_body,
        out_shape=jax.ShapeDtypeStruct((1, b), jnp.float32),
    )(acc)


def _sc_kernel(n_tok, b, ntp, nrp,
               seg_hbm, r_hbm, t_hbm, h_hbm, m_hbm, tsum_hbm, rsum_hbm,
               acc_hbm,
               h_v, tsum_v, rsum_v, seg_v, r_v, t_v,
               idx_v, s_v, ex_v, exd_v, zero_v,
               den_sp, num_sp,
               sem_in, sem_ga, sem_gb, sem_s):
    cid = lax.axis_index("c")
    sid = lax.axis_index("s")
    tid = cid * NS + sid
    rows_per_w = n_tok // (NC * NS * ROW)
    bpw = b // NS
    row0 = tid * rows_per_w
    grp = 8
    ngrp = rows_per_w // grp

    in_copies = [
        (seg_hbm.at[pl.ds(row0, rows_per_w)], seg_v),
        (r_hbm.at[pl.ds(row0, rows_per_w)], r_v),
        (t_hbm.at[pl.ds(row0, rows_per_w)], t_v),
        (h_hbm, h_v),
        (tsum_hbm, tsum_v),
        (rsum_hbm, rsum_v),
    ]
    for src, dst in in_copies:
        pltpu.async_copy(src, dst, sem_in)

    @pl.loop(0, bpw // L)
    def _zero(k):
        zero_v[pl.ds(k * L, L)] = jnp.zeros((L,), jnp.float32)

    pltpu.sync_copy(zero_v, den_sp.at[pl.ds(sid * bpw, bpw)])
    pltpu.sync_copy(zero_v, num_sp.at[pl.ds(sid * bpw, bpw)])

    for src, dst in in_copies:
        pltpu.make_async_copy(src, dst, sem_in).wait()

    plsc.subcore_barrier()

    @pl.loop(0, rows_per_w)
    def _idx(j):
        for u in range(ROW // L):
            sl = pl.ds(u * L, L)
            seg16 = seg_v[j, sl]
            r16 = r_v[j, sl]
            h16 = plsc.load_gather(h_v, [seg16])
            idx_v[j, sl] = h16 * nrp + r16

    gsems = (sem_ga, sem_gb)

    def _fire_gathers(g, sem):
        @pl.loop(g * grp, g * grp + grp)
        def _f(j):
            pltpu.async_copy(m_hbm.at[idx_v.at[j]], s_v.at[j], sem)

    def _drain_gathers(g, sem):
        @pl.loop(g * grp, g * grp + grp)
        def _d(j):
            pltpu.make_async_copy(m_hbm.at[idx_v.at[j]], s_v.at[j], sem).wait()

    _fire_gathers(0, gsems[0])
    for g in range(ngrp):
        if g + 1 < ngrp:
            _fire_gathers(g + 1, gsems[(g + 1) % 2])
        _drain_gathers(g, gsems[g % 2])

        @pl.loop(g * grp, g * grp + grp)
        def _compute(j):
            for u in range(ROW // L):
                sl = pl.ds(u * L, L)
                s16 = s_v[j, sl]
                t16 = t_v[j, sl]
                r16 = r_v[j, sl]
                d16 = (plsc.load_gather(tsum_v, [t16])
                       - plsc.load_gather(rsum_v, [r16]))
                ex16 = jnp.exp(s16)
                ex_v[j, sl] = ex16
                exd_v[j, sl] = ex16 * d16

        @pl.loop(g * grp, g * grp + grp)
        def _scatter(j):
            pltpu.async_copy(ex_v.at[j], den_sp.at[seg_v.at[j]], sem_s,
                             add=True)
            pltpu.async_copy(exd_v.at[j], num_sp.at[seg_v.at[j]], sem_s,
                             add=True)

    @pl.loop(0, rows_per_w)
    def _drain_s(j):
        pltpu.make_async_copy(ex_v.at[j], den_sp.at[seg_v.at[j]],
                              sem_s).wait()
        pltpu.make_async_copy(exd_v.at[j], num_sp.at[seg_v.at[j]],
                              sem_s).wait()

    plsc.subcore_barrier()

    pltpu.sync_copy(den_sp.at[pl.ds(sid * bpw, bpw)],
                    acc_hbm.at[2 * cid, pl.ds(sid * bpw, bpw)])
    pltpu.sync_copy(num_sp.at[pl.ds(sid * bpw, bpw)],
                    acc_hbm.at[2 * cid + 1, pl.ds(sid * bpw, bpw)])


def _sc_run(seg2, r2, t2, h, m_flat, tsum, rsum):
    n_rows = seg2.shape[0]
    n_tok = n_rows * ROW
    b = h.shape[0]
    ntp = tsum.shape[0]
    nrp = rsum.shape[0]
    bpw = b // NS
    mesh = plsc.VectorSubcoreMesh(
        core_axis_name="c", subcore_axis_name="s", num_cores=NC)
    rows_per_w = n_rows // (NC * NS)
    grid_kernel = pl.kernel(
        functools.partial(_sc_kernel, n_tok, b, ntp, nrp),
        out_type=jax.ShapeDtypeStruct((4, b), jnp.float32),
        mesh=mesh,
        compiler_params=pltpu.CompilerParams(needs_layout_passes=False),
        scratch_types=[
            pltpu.VMEM((b,), jnp.int32),
            pltpu.VMEM((ntp,), jnp.float32),
            pltpu.VMEM((nrp,), jnp.float32),
            pltpu.VMEM((rows_per_w, ROW), jnp.int32),
            pltpu.VMEM((rows_per_w, ROW), jnp.int32),
            pltpu.VMEM((rows_per_w, ROW), jnp.int32),
            pltpu.VMEM((rows_per_w, ROW), jnp.int32),
            pltpu.VMEM((rows_per_w, ROW), jnp.float32),
            pltpu.VMEM((rows_per_w, ROW), jnp.float32),
            pltpu.VMEM((rows_per_w, ROW), jnp.float32),
            pltpu.VMEM((bpw,), jnp.float32),
            pltpu.VMEM_SHARED((b,), jnp.float32),
            pltpu.VMEM_SHARED((b,), jnp.float32),
            pltpu.SemaphoreType.DMA,
            pltpu.SemaphoreType.DMA,
            pltpu.SemaphoreType.DMA,
            pltpu.SemaphoreType.DMA,
        ],
    )
    return grid_kernel(seg2, r2, t2, h, m_flat, tsum, rsum)


def kernel(h, r_flat, t_flat, segment_ids, H_table, R_table, T_table):
    b = h.shape[0]
    n = segment_ids.shape[0]
    dim = H_table.shape[1]
    nr = R_table.shape[0]
    nrp = 64

    r_pad = jnp.pad(R_table, ((0, nrp - nr), (0, 0)))
    m, ts, rs = _precompute(H_table, r_pad, T_table)

    seg2 = segment_ids.reshape(n // ROW, ROW)
    r2 = r_flat.reshape(n // ROW, ROW)
    t2 = t_flat.reshape(n // ROW, ROW)
    acc = _sc_run(seg2, r2, t2, h, m.reshape(-1), ts.reshape(-1),
                  rs.reshape(-1))
    v = _combine(acc)
    return jnp.broadcast_to(v.reshape(b, 1), (b, dim))
